# Initial kernel scaffold; baseline (speedup 1.0000x reference)
import jax
import jax.numpy as jnp
from jax.experimental import pallas as pl


def kernel(token_ids, subword_mask, table):
    raise NotImplementedError("write your pallas kernel here")


# speedup vs baseline: 2.4589x; 2.4589x over previous
import functools

import jax
import jax.numpy as jnp
from jax import lax
from jax.experimental import pallas as pl
from jax.experimental.pallas import tpu as pltpu
from jax.experimental.pallas import tpu_sc as plsc

NC, NS, L = 2, 16, 16
NW = NC * NS


def _make_sc_embed(BW, S, V, D, C):
    WPW = BW // NW
    RG = C * S
    G = WPW // C
    NDMA = RG // 128
    HALVES = D // L

    mesh = plsc.VectorSubcoreMesh(core_axis_name="c", subcore_axis_name="s")

    @functools.partial(
        pl.kernel,
        out_type=jax.ShapeDtypeStruct((BW, D), jnp.float32),
        mesh=mesh,
        scratch_types=[
            pltpu.VMEM((NDMA, 128), jnp.int32),
            pltpu.VMEM((RG,), jnp.float32),
            pltpu.VMEM((RG, D), jnp.float32),
            pltpu.VMEM((C, D), jnp.float32),
            pltpu.SemaphoreType.DMA,
        ],
        compiler_params=pltpu.CompilerParams(use_tc_tiling_on_sc=False),
    )
    def sc_embed(ids_hbm, mask_hbm, table_hbm, out_hbm,
                 idx_v, mask_v, rows_v, out_v, sem):
        wid = lax.axis_index("s") * NC + lax.axis_index("c")

        def group(g, carry):
            base_w = wid * WPW + g * C
            base_r128 = wid * (WPW * S // 128) + g * NDMA
            pltpu.sync_copy(ids_hbm.at[pl.ds(base_r128, NDMA), :], idx_v)
            pltpu.sync_copy(mask_hbm.at[pl.ds(base_w * S, RG)], mask_v)
            cps = [
                pltpu.async_copy(table_hbm.at[idx_v.at[j]],
                                 rows_v.at[pl.ds(j * 128, 128)], sem)
                for j in range(NDMA)
            ]
            for cp in cps:
                cp.wait()

            def pair(p, carry2):
                r0 = p * L
                mv = mask_v[pl.ds(r0, L)]
                for u in range(L // S):
                    for h in range(HALVES):
                        acc = jnp.zeros((L,), jnp.float32)
                        for s in range(S):
                            m = mv[u * S + s]
                            acc = acc + rows_v[r0 + u * S + s,
                                               pl.ds(h * L, L)] * m
                        out_v[p * (L // S) + u, pl.ds(h * L, L)] = acc
                return carry2

            lax.fori_loop(0, C * S // L, pair, 0)
            pltpu.sync_copy(out_v, out_hbm.at[pl.ds(base_w, C), :])
            return carry

        lax.fori_loop(0, G, group, 0)

    return sc_embed


def kernel(token_ids, subword_mask, table):
    B, W, S = token_ids.shape
    V, D = table.shape
    BW = B * W
    N = BW * S
    C = 64

    ids2d = token_ids.reshape(N // 128, 128).astype(jnp.int32)
    maskf = subword_mask.reshape(N).astype(jnp.float32)

    out = _make_sc_embed(BW, S, V, D, C)(ids2d, maskf, table)
    return out.reshape(B, W, D)


# speedup vs baseline: 3.0283x; 1.2316x over previous
import functools

import jax
import jax.numpy as jnp
from jax import lax
from jax.experimental import pallas as pl
from jax.experimental.pallas import tpu as pltpu
from jax.experimental.pallas import tpu_sc as plsc

NC, NS, L = 2, 16, 16
NW = NC * NS


def _make_sc_embed(BW, S, V, D, C):
    WPW = BW // NW
    RG = C * S
    G = WPW // C
    NDMA = RG // 128
    R128PW = WPW * S // 128
    HALVES = D // L
    WPV = L // S

    mesh = plsc.VectorSubcoreMesh(core_axis_name="c", subcore_axis_name="s")

    @functools.partial(
        pl.kernel,
        out_type=jax.ShapeDtypeStruct((BW, D), jnp.float32),
        mesh=mesh,
        scratch_types=[
            pltpu.VMEM((2, NDMA, 128), jnp.int32),
            pltpu.VMEM((2, RG), jnp.float32),
            pltpu.VMEM((2, RG, D), jnp.float32),
            pltpu.VMEM((2, C, D), jnp.float32),
            pltpu.SemaphoreType.DMA,
            pltpu.SemaphoreType.DMA,
            pltpu.SemaphoreType.DMA,
            pltpu.SemaphoreType.DMA,
        ],
        compiler_params=pltpu.CompilerParams(use_tc_tiling_on_sc=False),
    )
    def sc_embed(ids_hbm, mask_hbm, table_hbm, out_hbm,
                 idx_v, mask_v, rows_v, out_v,
                 sem_in, sem_g, sem_out0, sem_out1):
        wid = lax.axis_index("s") * NC + lax.axis_index("c")

        def in_copies(g, b):
            return (
                pltpu.make_async_copy(
                    ids_hbm.at[pl.ds(wid * R128PW + g * NDMA, NDMA), :],
                    idx_v.at[b], sem_in),
                pltpu.make_async_copy(
                    mask_hbm.at[pl.ds((wid * WPW + g * C) * S, RG)],
                    mask_v.at[b], sem_in),
            )

        def fire_in(g, b):
            for cp in in_copies(g, b):
                cp.start()

        def wait_in(g, b):
            for cp in in_copies(g, b):
                cp.wait()

        def gather_copies(b):
            return [
                pltpu.make_async_copy(
                    table_hbm.at[idx_v.at[b, j]],
                    rows_v.at[b, pl.ds(j * 128, 128)], sem_g)
                for j in range(NDMA)
            ]

        def fire_gathers(b):
            for cp in gather_copies(b):
                cp.start()

        def wait_gathers(b):
            for cp in gather_copies(b):
                cp.wait()

        def out_copy(g, b):
            return pltpu.make_async_copy(
                out_v.at[b], out_hbm.at[pl.ds(wid * WPW + g * C, C), :],
                sem_out0 if b == 0 else sem_out1)

        def compute(b):
            def pair(p, carry2):
                r0 = p * L
                mv = mask_v[b, pl.ds(r0, L)]
                for u in range(WPV):
                    for h in range(HALVES):
                        acc = jnp.zeros((L,), jnp.float32)
                        for s in range(S):
                            m = mv[u * S + s]
                            acc = acc + rows_v[b, r0 + u * S + s,
                                               pl.ds(h * L, L)] * m
                        out_v[b, p * WPV + u, pl.ds(h * L, L)] = acc
                return carry2

            lax.fori_loop(0, C * S // L, pair, 0)

        def group(g, b, wait_prev_out, next_gather, next_in):
            wait_gathers(b)
            if next_gather:
                wait_in(g + 1, 1 - b)
                fire_gathers(1 - b)
            if wait_prev_out:
                out_copy(g - 2, b).wait()
            compute(b)
            out_copy(g, b).start()
            if next_in:
                fire_in(g + 2, b)

        fire_in(0, 0)
        wait_in(0, 0)
        fire_gathers(0)
        fire_in(1, 1)

        group(0, 0, False, True, True)
        group(1, 1, False, True, True)

        def pipelined(k, carry):
            group(2 * k + 0, 0, True, True, True)
            group(2 * k + 1, 1, True, True, True)
            return carry

        lax.fori_loop(1, G // 2 - 1, pipelined, 0)

        group(G - 2, 0, True, True, False)
        group(G - 1, 1, True, False, False)

        out_copy(G - 2, 0).wait()
        out_copy(G - 1, 1).wait()

    return sc_embed


def kernel(token_ids, subword_mask, table):
    B, W, S = token_ids.shape
    V, D = table.shape
    BW = B * W
    N = BW * S
    C = 128

    ids2d = token_ids.reshape(N // 128, 128).astype(jnp.int32)
    maskf = subword_mask.reshape(N).astype(jnp.float32)

    out = _make_sc_embed(BW, S, V, D, C)(ids2d, maskf, table)
    return out.reshape(B, W, D)
